# baseline (speedup 1.0000x reference)
import functools

import jax
import jax.numpy as jnp
from jax import lax
from jax.experimental import pallas as pl
from jax.experimental.pallas import tpu as pltpu
from jax.experimental.pallas import tpu_sc as plsc

N = 10000
D = 128
H = 32
E = 320000

NPAD = 10240
NW = 32
K = 128
G = 8
NG = 10
EPW = NG * G * K
EPAD = NW * EPW
ROWS_PER_TILE = NPAD // 16
BM = 1024


@functools.lru_cache(maxsize=None)
def _make_segsum(C):
    mesh = plsc.VectorSubcoreMesh(core_axis_name="c", subcore_axis_name="s")

    @functools.partial(
        pl.kernel,
        out_type=jax.ShapeDtypeStruct((2, NPAD, C), jnp.float32),
        mesh=mesh,
        scratch_types=[
            pltpu.VMEM((NG, G * K), jnp.int32),
            pltpu.VMEM((NG, G * K), jnp.int32),
            pltpu.VMEM((2, G * K, C), jnp.float32),
            pltpu.VMEM_SHARED((NPAD, C), jnp.float32),
            pltpu.SemaphoreType.DMA,
            pltpu.SemaphoreType.DMA,
        ],
        compiler_params=pltpu.CompilerParams(use_tc_tiling_on_sc=False),
    )
    def seg(table_hbm, src_hbm, dst_hbm, zeros_hbm, out_hbm,
            src_v, dst_v, rows_v, acc_sh, sem0, sem1):
        c = lax.axis_index("c")
        s = lax.axis_index("s")
        w = c * 16 + s
        r0 = s * ROWS_PER_TILE
        pltpu.sync_copy(zeros_hbm.at[pl.ds(r0, ROWS_PER_TILE)],
                        acc_sh.at[pl.ds(r0, ROWS_PER_TILE)])
        pltpu.sync_copy(src_hbm.at[w], src_v)
        pltpu.sync_copy(dst_hbm.at[w], dst_v)
        plsc.subcore_barrier()

        sems = (sem0, sem1)
        pltpu.async_copy(table_hbm.at[src_v.at[0]], rows_v.at[0], sem0)
        pltpu.async_copy(table_hbm.at[src_v.at[1]], rows_v.at[1], sem1)
        for g in range(NG):
            b = g % 2
            pltpu.make_async_copy(table_hbm.at[src_v.at[g]],
                                  rows_v.at[b], sems[b]).wait()
            pltpu.sync_copy(rows_v.at[b], acc_sh.at[dst_v.at[g]], add=True)
            if g + 2 < NG:
                pltpu.async_copy(table_hbm.at[src_v.at[g + 2]],
                                 rows_v.at[b], sems[b])
        plsc.subcore_barrier()
        pltpu.sync_copy(acc_sh.at[pl.ds(r0, ROWS_PER_TILE)],
                        out_hbm.at[c, pl.ds(r0, ROWS_PER_TILE)])

    return seg


def _segsum32(table, src3, dst3, zeros):
    return _make_segsum(H)(table, src3, dst3, zeros)


def _segsum16(table, src3, dst3, zeros):
    return _make_segsum(16)(table, src3, dst3, zeros)


def _proj1(x_pad, W1a):
    def body(x_ref, w_ref, o_ref):
        o_ref[...] = jnp.dot(x_ref[...], w_ref[...],
                             preferred_element_type=jnp.float32)
    return pl.pallas_call(
        body,
        grid=(NPAD // BM,),
        in_specs=[pl.BlockSpec((BM, D), lambda i: (i, 0)),
                  pl.BlockSpec((D, H), lambda i: (0, 0))],
        out_specs=pl.BlockSpec((BM, H), lambda i: (i, 0)),
        out_shape=jax.ShapeDtypeStruct((NPAD, H), jnp.float32),
    )(x_pad, W1a)


def _mlp_step(q, sa, sb, b_in, Wmid, b_mid, Wout):
    CO = Wout.shape[1]

    def body(q_ref, sa_ref, sb_ref, bi_ref, wm_ref, bm_ref, wo_ref, o_ref):
        pre = q_ref[...] + sa_ref[...] + sb_ref[...] + bi_ref[...]
        h = jnp.dot(jnp.maximum(pre, 0.0), wm_ref[...],
                    preferred_element_type=jnp.float32) + bm_ref[...]
        o_ref[...] = jnp.dot(h, wo_ref[...],
                             preferred_element_type=jnp.float32)

    return pl.pallas_call(
        body,
        grid=(NPAD // BM,),
        in_specs=[pl.BlockSpec((BM, H), lambda i: (i, 0)),
                  pl.BlockSpec((BM, H), lambda i: (i, 0)),
                  pl.BlockSpec((BM, H), lambda i: (i, 0)),
                  pl.BlockSpec((1, H), lambda i: (0, 0)),
                  pl.BlockSpec((H, H), lambda i: (0, 0)),
                  pl.BlockSpec((1, H), lambda i: (0, 0)),
                  pl.BlockSpec((H, CO), lambda i: (0, 0))],
        out_specs=pl.BlockSpec((BM, CO), lambda i: (i, 0)),
        out_shape=jax.ShapeDtypeStruct((NPAD, CO), jnp.float32),
    )(q, sa, sb, b_in, Wmid, b_mid, Wout)


def _final_add(z16, sa, sb, b3):
    def body(z_ref, sa_ref, sb_ref, b3_ref, o_ref):
        o_ref[...] = (z_ref[:, :1] + sa_ref[:, :1] + sb_ref[:, :1]
                      + b3_ref[...])
    return pl.pallas_call(
        body,
        grid=(NPAD // BM,),
        in_specs=[pl.BlockSpec((BM, 16), lambda i: (i, 0)),
                  pl.BlockSpec((BM, 16), lambda i: (i, 0)),
                  pl.BlockSpec((BM, 16), lambda i: (i, 0)),
                  pl.BlockSpec((1, 1), lambda i: (0, 0))],
        out_specs=pl.BlockSpec((BM, 1), lambda i: (i, 0)),
        out_shape=jax.ShapeDtypeStruct((NPAD, 1), jnp.float32),
    )(z16, sa, sb, b3)


def kernel(x, edge_index, W1a, b1a, W1b, b1b, W2a, b2a, W2b, b2b, W3, b3):
    src = edge_index[0]
    dst = edge_index[1]
    pad = EPAD - E
    src3 = jnp.concatenate(
        [src, jnp.zeros((pad,), jnp.int32)]).reshape(NW, NG, G * K)
    dst3 = jnp.concatenate(
        [dst, jnp.full((pad,), N, jnp.int32)]).reshape(NW, NG, G * K)

    x_pad = jnp.pad(x, ((0, NPAD - N), (0, 0)))
    zeros32 = jnp.zeros((NPAD, H), jnp.float32)
    zeros16 = jnp.zeros((NPAD, 16), jnp.float32)
    W3p = jnp.pad(W3, ((0, 0), (0, 15)))

    q1 = _proj1(x_pad, W1a)
    s1 = _segsum32(q1, src3, dst3, zeros32)
    q2 = _mlp_step(q1, s1[0], s1[1], b1a.reshape(1, H), W1b,
                   b1b.reshape(1, H), W2a)
    s2 = _segsum32(q2, src3, dst3, zeros32)
    z16 = _mlp_step(q2, s2[0], s2[1], b2a.reshape(1, H), W2b,
                    b2b.reshape(1, H), W3p)
    s3 = _segsum16(z16, src3, dst3, zeros16)
    out = _final_add(z16, s3[0], s3[1], b3.reshape(1, 1))
    return out[:N]


# speedup vs baseline: 1.0178x; 1.0178x over previous
import functools

import jax
import jax.numpy as jnp
from jax import lax
from jax.experimental import pallas as pl
from jax.experimental.pallas import tpu as pltpu
from jax.experimental.pallas import tpu_sc as plsc

N = 10000
D = 128
H = 32
E = 320000

NPAD = 10240
NW = 32
K = 128
NBUF = 8
AHEAD = 4
NBATCH = 80
EPW = NBATCH * K
EPAD = NW * EPW
ROWS_PER_TILE = NPAD // 16
BM = 1024


@functools.lru_cache(maxsize=None)
def _make_segsum(C):
    mesh = plsc.VectorSubcoreMesh(core_axis_name="c", subcore_axis_name="s")

    @functools.partial(
        pl.kernel,
        out_type=jax.ShapeDtypeStruct((2, NPAD, C), jnp.float32),
        mesh=mesh,
        scratch_types=[
            pltpu.VMEM((NBATCH, K), jnp.int32),
            pltpu.VMEM((NBATCH, K), jnp.int32),
            pltpu.VMEM((NBUF, K, C), jnp.float32),
            pltpu.VMEM_SHARED((NPAD, C), jnp.float32),
            [pltpu.SemaphoreType.DMA] * NBUF,
            pltpu.SemaphoreType.DMA,
        ],
        compiler_params=pltpu.CompilerParams(use_tc_tiling_on_sc=False),
    )
    def seg(table_hbm, src_hbm, dst_hbm, zeros_hbm, out_hbm,
            src_v, dst_v, rows_v, acc_sh, gsems, ssem):
        c = lax.axis_index("c")
        s = lax.axis_index("s")
        w = c * 16 + s
        r0 = s * ROWS_PER_TILE
        pltpu.sync_copy(zeros_hbm.at[pl.ds(r0, ROWS_PER_TILE)],
                        acc_sh.at[pl.ds(r0, ROWS_PER_TILE)])
        pltpu.sync_copy(src_hbm.at[w], src_v)
        pltpu.sync_copy(dst_hbm.at[w], dst_v)
        plsc.subcore_barrier()

        def gather(g, b):
            pltpu.async_copy(table_hbm.at[src_v.at[g]], rows_v.at[b],
                             gsems[b])

        def wait_gather(g, b):
            pltpu.make_async_copy(table_hbm.at[src_v.at[g]], rows_v.at[b],
                                  gsems[b]).wait()

        def drain_scatter():
            pltpu.make_async_copy(rows_v.at[0], acc_sh.at[dst_v.at[0]],
                                  ssem).wait()

        for b in range(AHEAD):
            gather(b, b)

        def outer(i, _):
            g0 = i * NBUF
            for u in range(NBUF):
                g = g0 + u
                wait_gather(g, u)
                pltpu.async_copy(rows_v.at[u], acc_sh.at[dst_v.at[g]],
                                 ssem, add=True)

                @pl.when(g >= NBUF - AHEAD)
                def _():
                    drain_scatter()

                @pl.when(g + AHEAD < NBATCH)
                def _():
                    gather(g + AHEAD, (u + AHEAD) % NBUF)
            return 0

        lax.fori_loop(0, NBATCH // NBUF, outer, 0)
        for _ in range(AHEAD):
            drain_scatter()
        plsc.subcore_barrier()
        pltpu.sync_copy(acc_sh.at[pl.ds(r0, ROWS_PER_TILE)],
                        out_hbm.at[c, pl.ds(r0, ROWS_PER_TILE)])

    return seg


def _segsum32(table, src3, dst3, zeros):
    return _make_segsum(H)(table, src3, dst3, zeros)


def _segsum16(table, src3, dst3, zeros):
    return _make_segsum(16)(table, src3, dst3, zeros)


def _proj1(x_pad, W1a):
    def body(x_ref, w_ref, o_ref):
        o_ref[...] = jnp.dot(x_ref[...], w_ref[...],
                             preferred_element_type=jnp.float32)
    return pl.pallas_call(
        body,
        grid=(NPAD // BM,),
        in_specs=[pl.BlockSpec((BM, D), lambda i: (i, 0)),
                  pl.BlockSpec((D, H), lambda i: (0, 0))],
        out_specs=pl.BlockSpec((BM, H), lambda i: (i, 0)),
        out_shape=jax.ShapeDtypeStruct((NPAD, H), jnp.float32),
    )(x_pad, W1a)


def _mlp_step(q, sa, sb, b_in, Wmid, b_mid, Wout):
    CO = Wout.shape[1]

    def body(q_ref, sa_ref, sb_ref, bi_ref, wm_ref, bm_ref, wo_ref, o_ref):
        pre = q_ref[...] + sa_ref[...] + sb_ref[...] + bi_ref[...]
        h = jnp.dot(jnp.maximum(pre, 0.0), wm_ref[...],
                    preferred_element_type=jnp.float32) + bm_ref[...]
        o_ref[...] = jnp.dot(h, wo_ref[...],
                             preferred_element_type=jnp.float32)

    return pl.pallas_call(
        body,
        grid=(NPAD // BM,),
        in_specs=[pl.BlockSpec((BM, H), lambda i: (i, 0)),
                  pl.BlockSpec((BM, H), lambda i: (i, 0)),
                  pl.BlockSpec((BM, H), lambda i: (i, 0)),
                  pl.BlockSpec((1, H), lambda i: (0, 0)),
                  pl.BlockSpec((H, H), lambda i: (0, 0)),
                  pl.BlockSpec((1, H), lambda i: (0, 0)),
                  pl.BlockSpec((H, CO), lambda i: (0, 0))],
        out_specs=pl.BlockSpec((BM, CO), lambda i: (i, 0)),
        out_shape=jax.ShapeDtypeStruct((NPAD, CO), jnp.float32),
    )(q, sa, sb, b_in, Wmid, b_mid, Wout)


def _final_add(z16, sa, sb, b3):
    def body(z_ref, sa_ref, sb_ref, b3_ref, o_ref):
        o_ref[...] = (z_ref[:, :1] + sa_ref[:, :1] + sb_ref[:, :1]
                      + b3_ref[...])
    return pl.pallas_call(
        body,
        grid=(NPAD // BM,),
        in_specs=[pl.BlockSpec((BM, 16), lambda i: (i, 0)),
                  pl.BlockSpec((BM, 16), lambda i: (i, 0)),
                  pl.BlockSpec((BM, 16), lambda i: (i, 0)),
                  pl.BlockSpec((1, 1), lambda i: (0, 0))],
        out_specs=pl.BlockSpec((BM, 1), lambda i: (i, 0)),
        out_shape=jax.ShapeDtypeStruct((NPAD, 1), jnp.float32),
    )(z16, sa, sb, b3)


def kernel(x, edge_index, W1a, b1a, W1b, b1b, W2a, b2a, W2b, b2b, W3, b3):
    src = edge_index[0]
    dst = edge_index[1]
    pad = EPAD - E
    src3 = jnp.concatenate(
        [src, jnp.zeros((pad,), jnp.int32)]).reshape(NW, NBATCH, K)
    dst3 = jnp.concatenate(
        [dst, jnp.full((pad,), N, jnp.int32)]).reshape(NW, NBATCH, K)

    x_pad = jnp.pad(x, ((0, NPAD - N), (0, 0)))
    zeros32 = jnp.zeros((NPAD, H), jnp.float32)
    zeros16 = jnp.zeros((NPAD, 16), jnp.float32)
    W3p = jnp.pad(W3, ((0, 0), (0, 15)))

    q1 = _proj1(x_pad, W1a)
    s1 = _segsum32(q1, src3, dst3, zeros32)
    q2 = _mlp_step(q1, s1[0], s1[1], b1a.reshape(1, H), W1b,
                   b1b.reshape(1, H), W2a)
    s2 = _segsum32(q2, src3, dst3, zeros32)
    z16 = _mlp_step(q2, s2[0], s2[1], b2a.reshape(1, H), W2b,
                    b2b.reshape(1, H), W3p)
    s3 = _segsum16(z16, src3, dst3, zeros16)
    out = _final_add(z16, s3[0], s3[1], b3.reshape(1, 1))
    return out[:N]


# speedup vs baseline: 1.0186x; 1.0008x over previous
import functools

import jax
import jax.numpy as jnp
from jax import lax
from jax.experimental import pallas as pl
from jax.experimental.pallas import tpu as pltpu
from jax.experimental.pallas import tpu_sc as plsc

N = 10000
D = 128
H = 32
E = 320000

NPAD = 10240
NW = 32
K = 128
NBUF = 8
AHEAD = 4
NBATCH = 80
EPW = NBATCH * K
EPAD = NW * EPW
ROWS_PER_TILE = NPAD // 16
BM = 1024


@functools.lru_cache(maxsize=None)
def _make_segsum(C):
    mesh = plsc.VectorSubcoreMesh(core_axis_name="c", subcore_axis_name="s")

    @functools.partial(
        pl.kernel,
        out_type=jax.ShapeDtypeStruct((2, NPAD, C), jnp.float32),
        mesh=mesh,
        scratch_types=[
            pltpu.VMEM((NBATCH, K), jnp.int32),
            pltpu.VMEM((NBATCH, K), jnp.int32),
            pltpu.VMEM((2, K, C), jnp.float32),
            pltpu.VMEM_SHARED((NPAD, C), jnp.float32),
            [pltpu.SemaphoreType.DMA] * 2,
            pltpu.SemaphoreType.DMA,
        ],
        compiler_params=pltpu.CompilerParams(
            use_tc_tiling_on_sc=False, skip_device_barrier=True),
    )
    def seg(table_hbm, src_hbm, dst_hbm, zeros_hbm, out_hbm,
            src_v, dst_v, rows_v, acc_sh, gsems, ssem):
        c = lax.axis_index("c")
        s = lax.axis_index("s")
        w = c * 16 + s
        r0 = s * ROWS_PER_TILE
        pltpu.sync_copy(zeros_hbm.at[pl.ds(r0, ROWS_PER_TILE)],
                        acc_sh.at[pl.ds(r0, ROWS_PER_TILE)])
        pltpu.sync_copy(src_hbm.at[w], src_v)
        pltpu.sync_copy(dst_hbm.at[w], dst_v)
        plsc.subcore_barrier()

        pltpu.async_copy(table_hbm.at[src_v.at[0]], rows_v.at[0], gsems[0])

        def body2(i, _):
            j0 = i * 2

            pltpu.async_copy(table_hbm.at[src_v.at[j0 + 1]],
                             rows_v.at[1], gsems[1])
            pltpu.make_async_copy(table_hbm.at[src_v.at[j0]],
                                  rows_v.at[0], gsems[0]).wait()
            pltpu.sync_copy(rows_v.at[0], acc_sh.at[dst_v.at[j0]], add=True)

            @pl.when(j0 + 2 < NBATCH)
            def _g2():
                pltpu.async_copy(table_hbm.at[src_v.at[j0 + 2]],
                                 rows_v.at[0], gsems[0])

            pltpu.make_async_copy(table_hbm.at[src_v.at[j0 + 1]],
                                  rows_v.at[1], gsems[1]).wait()
            pltpu.sync_copy(rows_v.at[1], acc_sh.at[dst_v.at[j0 + 1]],
                            add=True)
            return 0

        lax.fori_loop(0, NBATCH // 2, body2, 0)
        plsc.subcore_barrier()
        pltpu.sync_copy(acc_sh.at[pl.ds(r0, ROWS_PER_TILE)],
                        out_hbm.at[c, pl.ds(r0, ROWS_PER_TILE)])

    return seg


def _segsum32(table, src3, dst3, zeros):
    return _make_segsum(H)(table, src3, dst3, zeros)


def _segsum16(table, src3, dst3, zeros):
    return _make_segsum(16)(table, src3, dst3, zeros)


def _proj1(x_pad, W1a):
    def body(x_ref, w_ref, o_ref):
        o_ref[...] = jnp.dot(x_ref[...], w_ref[...],
                             preferred_element_type=jnp.float32)
    return pl.pallas_call(
        body,
        grid=(NPAD // BM,),
        in_specs=[pl.BlockSpec((BM, D), lambda i: (i, 0)),
                  pl.BlockSpec((D, H), lambda i: (0, 0))],
        out_specs=pl.BlockSpec((BM, H), lambda i: (i, 0)),
        out_shape=jax.ShapeDtypeStruct((NPAD, H), jnp.float32),
    )(x_pad, W1a)


def _mlp_step(q, sa, sb, b_in, Wmid, b_mid, Wout):
    CO = Wout.shape[1]

    def body(q_ref, sa_ref, sb_ref, bi_ref, wm_ref, bm_ref, wo_ref, o_ref):
        pre = q_ref[...] + sa_ref[...] + sb_ref[...] + bi_ref[...]
        h = jnp.dot(jnp.maximum(pre, 0.0), wm_ref[...],
                    preferred_element_type=jnp.float32) + bm_ref[...]
        o_ref[...] = jnp.dot(h, wo_ref[...],
                             preferred_element_type=jnp.float32)

    return pl.pallas_call(
        body,
        grid=(NPAD // BM,),
        in_specs=[pl.BlockSpec((BM, H), lambda i: (i, 0)),
                  pl.BlockSpec((BM, H), lambda i: (i, 0)),
                  pl.BlockSpec((BM, H), lambda i: (i, 0)),
                  pl.BlockSpec((1, H), lambda i: (0, 0)),
                  pl.BlockSpec((H, H), lambda i: (0, 0)),
                  pl.BlockSpec((1, H), lambda i: (0, 0)),
                  pl.BlockSpec((H, CO), lambda i: (0, 0))],
        out_specs=pl.BlockSpec((BM, CO), lambda i: (i, 0)),
        out_shape=jax.ShapeDtypeStruct((NPAD, CO), jnp.float32),
    )(q, sa, sb, b_in, Wmid, b_mid, Wout)


def _final_add(z16, sa, sb, b3):
    def body(z_ref, sa_ref, sb_ref, b3_ref, o_ref):
        o_ref[...] = (z_ref[:, :1] + sa_ref[:, :1] + sb_ref[:, :1]
                      + b3_ref[...])
    return pl.pallas_call(
        body,
        grid=(NPAD // BM,),
        in_specs=[pl.BlockSpec((BM, 16), lambda i: (i, 0)),
                  pl.BlockSpec((BM, 16), lambda i: (i, 0)),
                  pl.BlockSpec((BM, 16), lambda i: (i, 0)),
                  pl.BlockSpec((1, 1), lambda i: (0, 0))],
        out_specs=pl.BlockSpec((BM, 1), lambda i: (i, 0)),
        out_shape=jax.ShapeDtypeStruct((NPAD, 1), jnp.float32),
    )(z16, sa, sb, b3)


def kernel(x, edge_index, W1a, b1a, W1b, b1b, W2a, b2a, W2b, b2b, W3, b3):
    src = edge_index[0]
    dst = edge_index[1]
    pad = EPAD - E
    src3 = jnp.concatenate(
        [src, jnp.zeros((pad,), jnp.int32)]).reshape(NW, NBATCH, K)
    dst3 = jnp.concatenate(
        [dst, jnp.full((pad,), N, jnp.int32)]).reshape(NW, NBATCH, K)

    x_pad = jnp.pad(x, ((0, NPAD - N), (0, 0)))
    zeros32 = jnp.zeros((NPAD, H), jnp.float32)
    zeros16 = jnp.zeros((NPAD, 16), jnp.float32)
    W3p = jnp.pad(W3, ((0, 0), (0, 15)))

    q1 = _proj1(x_pad, W1a)
    s1 = _segsum32(q1, src3, dst3, zeros32)
    q2 = _mlp_step(q1, s1[0], s1[1], b1a.reshape(1, H), W1b,
                   b1b.reshape(1, H), W2a)
    s2 = _segsum32(q2, src3, dst3, zeros32)
    z16 = _mlp_step(q2, s2[0], s2[1], b2a.reshape(1, H), W2b,
                    b2b.reshape(1, H), W3p)
    s3 = _segsum16(z16, src3, dst3, zeros16)
    out = _final_add(z16, s3[0], s3[1], b3.reshape(1, 1))
    return out[:N]


# speedup vs baseline: 1.0190x; 1.0004x over previous
import functools

import jax
import jax.numpy as jnp
from jax import lax
from jax.experimental import pallas as pl
from jax.experimental.pallas import tpu as pltpu
from jax.experimental.pallas import tpu_sc as plsc

N = 10000
D = 128
H = 32
E = 320000

NPAD = 10240
NW = 32
K = 128
NBUF = 8
AHEAD = 4
NBATCH = 80
EPW = NBATCH * K
EPAD = NW * EPW
ROWS_PER_TILE = NPAD // 16
BM = 1024


@functools.lru_cache(maxsize=None)
def _make_segsum(C):
    mesh = plsc.VectorSubcoreMesh(core_axis_name="c", subcore_axis_name="s")

    @functools.partial(
        pl.kernel,
        out_type=jax.ShapeDtypeStruct((2, NPAD, C), jnp.float32),
        mesh=mesh,
        scratch_types=[
            pltpu.VMEM((NBATCH, K), jnp.int32),
            pltpu.VMEM((NBATCH, K), jnp.int32),
            pltpu.VMEM((2, K, C), jnp.float32),
            pltpu.VMEM_SHARED((NPAD, C), jnp.float32),
            [pltpu.SemaphoreType.DMA] * 2,
            pltpu.SemaphoreType.DMA,
        ],
        compiler_params=pltpu.CompilerParams(use_tc_tiling_on_sc=False),
    )
    def seg(table_hbm, src_hbm, dst_hbm, zeros_hbm, out_hbm,
            src_v, dst_v, rows_v, acc_sh, gsems, ssem):
        c = lax.axis_index("c")
        s = lax.axis_index("s")
        w = c * 16 + s
        r0 = s * ROWS_PER_TILE
        pltpu.sync_copy(zeros_hbm.at[pl.ds(r0, ROWS_PER_TILE)],
                        acc_sh.at[pl.ds(r0, ROWS_PER_TILE)])
        pltpu.sync_copy(src_hbm.at[w], src_v)
        pltpu.sync_copy(dst_hbm.at[w], dst_v)
        plsc.subcore_barrier()

        pltpu.async_copy(table_hbm.at[src_v.at[0]], rows_v.at[0], gsems[0])

        def body2(i, _):
            j0 = i * 2

            pltpu.async_copy(table_hbm.at[src_v.at[j0 + 1]],
                             rows_v.at[1], gsems[1])
            pltpu.make_async_copy(table_hbm.at[src_v.at[j0]],
                                  rows_v.at[0], gsems[0]).wait()
            pltpu.sync_copy(rows_v.at[0], acc_sh.at[dst_v.at[j0]], add=True)

            @pl.when(j0 + 2 < NBATCH)
            def _g2():
                pltpu.async_copy(table_hbm.at[src_v.at[j0 + 2]],
                                 rows_v.at[0], gsems[0])

            pltpu.make_async_copy(table_hbm.at[src_v.at[j0 + 1]],
                                  rows_v.at[1], gsems[1]).wait()
            pltpu.sync_copy(rows_v.at[1], acc_sh.at[dst_v.at[j0 + 1]],
                            add=True)
            return 0

        lax.fori_loop(0, NBATCH // 2, body2, 0)
        plsc.subcore_barrier()
        pltpu.sync_copy(acc_sh.at[pl.ds(r0, ROWS_PER_TILE)],
                        out_hbm.at[c, pl.ds(r0, ROWS_PER_TILE)])

    return seg


def _segsum32(table, src3, dst3, zeros):
    return _make_segsum(H)(table, src3, dst3, zeros)


def _segsum16(table, src3, dst3, zeros):
    return _make_segsum(16)(table, src3, dst3, zeros)


def _proj1(x_pad, W1a):
    def body(x_ref, w_ref, o_ref):
        o_ref[...] = jnp.dot(x_ref[...], w_ref[...],
                             preferred_element_type=jnp.float32)
    return pl.pallas_call(
        body,
        grid=(NPAD // BM,),
        in_specs=[pl.BlockSpec((BM, D), lambda i: (i, 0)),
                  pl.BlockSpec((D, H), lambda i: (0, 0))],
        out_specs=pl.BlockSpec((BM, H), lambda i: (i, 0)),
        out_shape=jax.ShapeDtypeStruct((NPAD, H), jnp.float32),
    )(x_pad, W1a)


def _mlp_step(q, sa, sb, b_in, Wmid, b_mid, Wout):
    CO = Wout.shape[1]

    def body(q_ref, sa_ref, sb_ref, bi_ref, wm_ref, bm_ref, wo_ref, o_ref):
        pre = q_ref[...] + sa_ref[...] + sb_ref[...] + bi_ref[...]
        h = jnp.dot(jnp.maximum(pre, 0.0), wm_ref[...],
                    preferred_element_type=jnp.float32) + bm_ref[...]
        o_ref[...] = jnp.dot(h, wo_ref[...],
                             preferred_element_type=jnp.float32)

    return pl.pallas_call(
        body,
        grid=(NPAD // BM,),
        in_specs=[pl.BlockSpec((BM, H), lambda i: (i, 0)),
                  pl.BlockSpec((BM, H), lambda i: (i, 0)),
                  pl.BlockSpec((BM, H), lambda i: (i, 0)),
                  pl.BlockSpec((1, H), lambda i: (0, 0)),
                  pl.BlockSpec((H, H), lambda i: (0, 0)),
                  pl.BlockSpec((1, H), lambda i: (0, 0)),
                  pl.BlockSpec((H, CO), lambda i: (0, 0))],
        out_specs=pl.BlockSpec((BM, CO), lambda i: (i, 0)),
        out_shape=jax.ShapeDtypeStruct((NPAD, CO), jnp.float32),
    )(q, sa, sb, b_in, Wmid, b_mid, Wout)


def _final_add(z16, sa, sb, b3):
    def body(z_ref, sa_ref, sb_ref, b3_ref, o_ref):
        o_ref[...] = (z_ref[:, :1] + sa_ref[:, :1] + sb_ref[:, :1]
                      + b3_ref[...])
    return pl.pallas_call(
        body,
        grid=(NPAD // BM,),
        in_specs=[pl.BlockSpec((BM, 16), lambda i: (i, 0)),
                  pl.BlockSpec((BM, 16), lambda i: (i, 0)),
                  pl.BlockSpec((BM, 16), lambda i: (i, 0)),
                  pl.BlockSpec((1, 1), lambda i: (0, 0))],
        out_specs=pl.BlockSpec((BM, 1), lambda i: (i, 0)),
        out_shape=jax.ShapeDtypeStruct((NPAD, 1), jnp.float32),
    )(z16, sa, sb, b3)


def kernel(x, edge_index, W1a, b1a, W1b, b1b, W2a, b2a, W2b, b2b, W3, b3):
    src = edge_index[0]
    dst = edge_index[1]
    pad = EPAD - E
    src3 = jnp.concatenate(
        [src, jnp.zeros((pad,), jnp.int32)]).reshape(NW, NBATCH, K)
    dst3 = jnp.concatenate(
        [dst, jnp.full((pad,), N, jnp.int32)]).reshape(NW, NBATCH, K)

    x_pad = jnp.pad(x, ((0, NPAD - N), (0, 0)))
    zeros32 = jnp.zeros((NPAD, H), jnp.float32)
    zeros16 = jnp.zeros((NPAD, 16), jnp.float32)
    W3p = jnp.pad(W3, ((0, 0), (0, 15)))

    q1 = _proj1(x_pad, W1a)
    s1 = _segsum32(q1, src3, dst3, zeros32)
    q2 = _mlp_step(q1, s1[0], s1[1], b1a.reshape(1, H), W1b,
                   b1b.reshape(1, H), W2a)
    s2 = _segsum32(q2, src3, dst3, zeros32)
    z16 = _mlp_step(q2, s2[0], s2[1], b2a.reshape(1, H), W2b,
                    b2b.reshape(1, H), W3p)
    s3 = _segsum16(z16, src3, dst3, zeros16)
    out = _final_add(z16, s3[0], s3[1], b3.reshape(1, 1))
    return out[:N]


# speedup vs baseline: 1.2669x; 1.2433x over previous
import functools

import jax
import jax.numpy as jnp
from jax import lax
from jax.experimental import pallas as pl
from jax.experimental.pallas import tpu as pltpu
from jax.experimental.pallas import tpu_sc as plsc

N = 10000
D = 128
H = 32
E = 320000

NPAD = 10240
NW = 32
K = 128
NBATCH = 79
EPW = NBATCH * K
EPAD = NW * EPW
ROWS_PER_TILE = NPAD // 16
BM = 1024


@functools.lru_cache(maxsize=None)
def _make_segsum(C):
    mesh = plsc.VectorSubcoreMesh(core_axis_name="c", subcore_axis_name="s")

    @functools.partial(
        pl.kernel,
        out_type=jax.ShapeDtypeStruct((2, NPAD, C), jnp.float32),
        mesh=mesh,
        scratch_types=[
            pltpu.VMEM((NBATCH, K), jnp.int32),
            pltpu.VMEM((NBATCH, K), jnp.int32),
            pltpu.VMEM((2, K, C), jnp.float32),
            pltpu.VMEM_SHARED((NPAD, C), jnp.float32),
            pltpu.SemaphoreType.DMA,
            pltpu.SemaphoreType.DMA,
        ],
        compiler_params=pltpu.CompilerParams(use_tc_tiling_on_sc=False),
    )
    def seg(table_hbm, src_hbm, dst_hbm, zeros_hbm, out_hbm,
            src_v, dst_v, rows_v, acc_sh, sem0, sem1):
        c = lax.axis_index("c")
        s = lax.axis_index("s")
        w = c * 16 + s
        r0 = s * ROWS_PER_TILE
        pltpu.sync_copy(zeros_hbm.at[pl.ds(r0, ROWS_PER_TILE)],
                        acc_sh.at[pl.ds(r0, ROWS_PER_TILE)])
        pltpu.sync_copy(src_hbm.at[w], src_v)
        pltpu.sync_copy(dst_hbm.at[w], dst_v)
        plsc.subcore_barrier()

        pltpu.async_copy(table_hbm.at[src_v.at[0]], rows_v.at[0], sem0)

        def body2(i, _):
            j0 = i * 2

            @pl.when(j0 + 1 < NBATCH)
            def _g1():
                pltpu.async_copy(table_hbm.at[src_v.at[j0 + 1]],
                                 rows_v.at[1], sem1)
            pltpu.make_async_copy(table_hbm.at[src_v.at[j0]],
                                  rows_v.at[0], sem0).wait()
            pltpu.sync_copy(rows_v.at[0], acc_sh.at[dst_v.at[j0]], add=True)

            @pl.when(j0 + 2 < NBATCH)
            def _g2():
                pltpu.async_copy(table_hbm.at[src_v.at[j0 + 2]],
                                 rows_v.at[0], sem0)

            @pl.when(j0 + 1 < NBATCH)
            def _s2():
                pltpu.make_async_copy(table_hbm.at[src_v.at[j0 + 1]],
                                      rows_v.at[1], sem1).wait()
                pltpu.sync_copy(rows_v.at[1], acc_sh.at[dst_v.at[j0 + 1]],
                                add=True)
            return 0

        lax.fori_loop(0, (NBATCH + 1) // 2, body2, 0)
        plsc.subcore_barrier()
        pltpu.sync_copy(acc_sh.at[pl.ds(r0, ROWS_PER_TILE)],
                        out_hbm.at[c, pl.ds(r0, ROWS_PER_TILE)])

    return seg


def _segsum32(table, src3, dst3, zeros):
    return _make_segsum(H)(table, src3, dst3, zeros)


def _segsum16(table, src3, dst3, zeros):
    return _make_segsum(16)(table, src3, dst3, zeros)


def _proj1(x_pad, W1a):
    def body(x_ref, w_ref, o_ref):
        o_ref[...] = jnp.dot(x_ref[...], w_ref[...],
                             preferred_element_type=jnp.float32)
    return pl.pallas_call(
        body,
        grid=(NPAD // BM,),
        in_specs=[pl.BlockSpec((BM, D), lambda i: (i, 0)),
                  pl.BlockSpec((D, H), lambda i: (0, 0))],
        out_specs=pl.BlockSpec((BM, H), lambda i: (i, 0)),
        out_shape=jax.ShapeDtypeStruct((NPAD, H), jnp.float32),
    )(x_pad, W1a)


def _mlp_step(q, sa, sb, b_in, Wmid, b_mid, Wout):
    CO = Wout.shape[1]

    def body(q_ref, sa_ref, sb_ref, bi_ref, wm_ref, bm_ref, wo_ref, o_ref):
        pre = q_ref[...] + sa_ref[...] + sb_ref[...] + bi_ref[...]
        h = jnp.dot(jnp.maximum(pre, 0.0), wm_ref[...],
                    preferred_element_type=jnp.float32) + bm_ref[...]
        o_ref[...] = jnp.dot(h, wo_ref[...],
                             preferred_element_type=jnp.float32)

    return pl.pallas_call(
        body,
        grid=(NPAD // BM,),
        in_specs=[pl.BlockSpec((BM, H), lambda i: (i, 0)),
                  pl.BlockSpec((BM, H), lambda i: (i, 0)),
                  pl.BlockSpec((BM, H), lambda i: (i, 0)),
                  pl.BlockSpec((1, H), lambda i: (0, 0)),
                  pl.BlockSpec((H, H), lambda i: (0, 0)),
                  pl.BlockSpec((1, H), lambda i: (0, 0)),
                  pl.BlockSpec((H, CO), lambda i: (0, 0))],
        out_specs=pl.BlockSpec((BM, CO), lambda i: (i, 0)),
        out_shape=jax.ShapeDtypeStruct((NPAD, CO), jnp.float32),
    )(q, sa, sb, b_in, Wmid, b_mid, Wout)


def _final_add(z16, sa, sb, b3):
    def body(z_ref, sa_ref, sb_ref, b3_ref, o_ref):
        o_ref[...] = (z_ref[:, :1] + sa_ref[:, :1] + sb_ref[:, :1]
                      + b3_ref[...])
    return pl.pallas_call(
        body,
        grid=(NPAD // BM,),
        in_specs=[pl.BlockSpec((BM, 16), lambda i: (i, 0)),
                  pl.BlockSpec((BM, 16), lambda i: (i, 0)),
                  pl.BlockSpec((BM, 16), lambda i: (i, 0)),
                  pl.BlockSpec((1, 1), lambda i: (0, 0))],
        out_specs=pl.BlockSpec((BM, 1), lambda i: (i, 0)),
        out_shape=jax.ShapeDtypeStruct((NPAD, 1), jnp.float32),
    )(z16, sa, sb, b3)


def kernel(x, edge_index, W1a, b1a, W1b, b1b, W2a, b2a, W2b, b2b, W3, b3):
    src = edge_index[0]
    dst = edge_index[1]
    pad = EPAD - E
    src3 = jnp.concatenate(
        [src, jnp.zeros((pad,), jnp.int32)]).reshape(NW, NBATCH, K)
    dst3 = jnp.concatenate(
        [dst, jnp.full((pad,), N, jnp.int32)]).reshape(NW, NBATCH, K)

    x_pad = jnp.pad(x, ((0, NPAD - N), (0, 0)))
    zeros32 = jnp.zeros((NPAD, H), jnp.float32)
    zeros16 = jnp.zeros((NPAD, 16), jnp.float32)
    W3p = jnp.pad(W3, ((0, 0), (0, 15)))

    q1 = _proj1(x_pad, W1a)
    s1 = _segsum32(q1, src3, dst3, zeros32)
    q2 = _mlp_step(q1, s1[0], s1[1], b1a.reshape(1, H), W1b,
                   b1b.reshape(1, H), W2a)
    s2 = _segsum32(q2, src3, dst3, zeros32)
    z16 = _mlp_step(q2, s2[0], s2[1], b2a.reshape(1, H), W2b,
                    b2b.reshape(1, H), W3p)
    s3 = _segsum16(z16, src3, dst3, zeros16)
    out = _final_add(z16, s3[0], s3[1], b3.reshape(1, 1))
    return out[:N]
